# baseline (speedup 1.0000x reference)
import functools

import jax
import jax.numpy as jnp
from jax.experimental import pallas as pl

_N = 4096
_F = 256
_C = 40
_NB1 = 25
_NB2 = 10
_BM = 256
_CK = 256


def _lrelu(x):
    return jnp.where(x >= 0, x, 0.01 * x)


def _dot(a, b):
    return jax.lax.dot_general(a, b, (((1,), (0,)), ((), ())),
                               preferred_element_type=jnp.float32)


def _split(x):
    hi = x.astype(jnp.bfloat16)
    lo = (x - hi.astype(jnp.float32)).astype(jnp.bfloat16)
    return hi, lo


def _dot_f32_via_bf16(a, b):
    ahi, alo = _split(a)
    bhi, blo = _split(b)
    return _dot(ahi, bhi) + (_dot(ahi, blo) + _dot(alo, bhi))


def _layer1_body(a_ref, xhi_ref, xlo_ref, xb_ref, wnT_ref, bn_ref, wT_ref,
                 b_ref, h_ref, sel2_ref, cnt_ref):
    a = a_ref[...]
    r = jax.lax.broadcasted_iota(jnp.int32, (_CK, _CK), 0)
    c = jax.lax.broadcasted_iota(jnp.int32, (_CK, _CK), 1)
    tri = (r <= c).astype(jnp.bfloat16)
    carry = jnp.zeros((_BM, 1), jnp.float32)
    acc = jnp.zeros((_BM, _F), jnp.float32)
    for ci in range(_N // _CK):
        sl = slice(ci * _CK, (ci + 1) * _CK)
        mcf = jnp.where(a[:, sl] != 0, 1.0, 0.0)
        csum = _dot(mcf.astype(jnp.bfloat16), tri) + carry
        lim1 = jnp.where(csum <= _NB1, 1.0, 0.0)
        lim2 = jnp.where(csum <= _NB2, 1.0, 0.0)
        sel1 = (mcf * lim1).astype(jnp.bfloat16)
        sel2_ref[:, sl] = (mcf * lim2).astype(jnp.int8)
        acc = acc + _dot(sel1, xhi_ref[pl.ds(ci * _CK, _CK), :])
        acc = acc + _dot(sel1, xlo_ref[pl.ds(ci * _CK, _CK), :])
        carry = carry + jnp.sum(mcf, axis=1, keepdims=True)
    cnt = jnp.minimum(carry, float(_NB1))
    mean = acc / jnp.maximum(cnt, 1.0)
    xj = _lrelu(_dot_f32_via_bf16(mean, wnT_ref[...]) + bn_ref[...])
    xi = _lrelu(_dot_f32_via_bf16(xb_ref[...], wT_ref[...]) + b_ref[...])
    h_ref[...] = xi + jnp.where(carry > 0, xj, 0.0)
    cnt_ref[...] = carry


def _layer2_body(sel2_ref, hhi_ref, hlo_ref, hb_ref, cnt_ref, wnT_ref, bn_ref,
                 wT_ref, b_ref, w3T_ref, b3_ref, o_ref):
    total = cnt_ref[...]
    acc = jnp.zeros((_BM, _F), jnp.float32)
    for ci in range(_N // _CK):
        sel2 = sel2_ref[:, ci * _CK:(ci + 1) * _CK].astype(jnp.bfloat16)
        acc = acc + _dot(sel2, hhi_ref[pl.ds(ci * _CK, _CK), :])
        acc = acc + _dot(sel2, hlo_ref[pl.ds(ci * _CK, _CK), :])
    cnt = jnp.minimum(total, float(_NB2))
    mean = acc / jnp.maximum(cnt, 1.0)
    xj = _lrelu(_dot_f32_via_bf16(mean, wnT_ref[...]) + bn_ref[...])
    xi = _lrelu(_dot_f32_via_bf16(hb_ref[...], wT_ref[...]) + b_ref[...])
    h2 = xi + jnp.where(total > 0, xj, 0.0)
    logits = _dot_f32_via_bf16(h2, w3T_ref[...]) + b3_ref[...]
    m = jnp.max(logits, axis=1, keepdims=True)
    shifted = logits - m
    lse = jnp.log(jnp.sum(jnp.exp(shifted), axis=1, keepdims=True))
    o_ref[...] = shifted - lse


def _full(shape):
    return pl.BlockSpec(shape, lambda i: (0, 0))


def kernel(X, A, Wn1, bn1, W1, b1, Wn2, bn2, W2, b2, W3, b3):
    grid = (_N // _BM,)
    row_block = lambda i: (i, 0)
    Xhi, Xlo = _split(X)

    h, sel2, cnt = pl.pallas_call(
        _layer1_body,
        grid=grid,
        in_specs=[
            pl.BlockSpec((_BM, _N), row_block),
            _full((_N, _F)),
            _full((_N, _F)),
            pl.BlockSpec((_BM, _F), row_block),
            _full((_F, _F)),
            _full((1, _F)),
            _full((_F, _F)),
            _full((1, _F)),
        ],
        out_specs=[
            pl.BlockSpec((_BM, _F), row_block),
            pl.BlockSpec((_BM, _N), row_block),
            pl.BlockSpec((_BM, 1), row_block),
        ],
        out_shape=[
            jax.ShapeDtypeStruct((_N, _F), jnp.float32),
            jax.ShapeDtypeStruct((_N, _N), jnp.int8),
            jax.ShapeDtypeStruct((_N, 1), jnp.float32),
        ],
    )(A, Xhi, Xlo, X, Wn1.T, bn1[None, :], W1.T, b1[None, :])

    hhi, hlo = _split(h)
    out = pl.pallas_call(
        _layer2_body,
        grid=grid,
        in_specs=[
            pl.BlockSpec((_BM, _N), row_block),
            _full((_N, _F)),
            _full((_N, _F)),
            pl.BlockSpec((_BM, _F), row_block),
            pl.BlockSpec((_BM, 1), row_block),
            _full((_F, _F)),
            _full((1, _F)),
            _full((_F, _F)),
            _full((1, _F)),
            _full((_F, _C)),
            _full((1, _C)),
        ],
        out_specs=pl.BlockSpec((_BM, _C), row_block),
        out_shape=jax.ShapeDtypeStruct((_N, _C), jnp.float32),
    )(sel2, hhi, hlo, h, cnt, Wn2.T, bn2[None, :], W2.T, b2[None, :],
      W3.T, b3[None, :])
    return out
